# Initial kernel scaffold; baseline (speedup 1.0000x reference)
import functools

import jax
import jax.numpy as jnp
from jax import lax
from jax.experimental import pallas as pl
from jax.experimental.pallas import tpu as pltpu
from jax.experimental.pallas import tpu_sc as plsc

N_ROWS = 16384
N_MEM = 100000
D = 128
_L = 16
_NW = 32
_B_W = N_ROWS // _NW
_CH = 64
_PCH = 2048


def _norm_body(val_ref, out_ref):
    v = val_ref[...] * (1.0 / 255.0)
    mean = jnp.mean(v, axis=0, keepdims=True)
    mn = jnp.min(v, axis=0, keepdims=True)
    mx = jnp.max(v, axis=0, keepdims=True)
    out_ref[...] = (v - mean) / jnp.abs(mx - mn)


def _normalize(val):
    return pl.pallas_call(
        _norm_body,
        out_shape=jax.ShapeDtypeStruct((N_ROWS, D), jnp.float32),
    )(val)


_mesh = plsc.VectorSubcoreMesh(core_axis_name="c", subcore_axis_name="s")


@functools.partial(
    pl.kernel,
    mesh=_mesh,
    out_type=jax.ShapeDtypeStruct((N_ROWS,), jnp.int32),
    scratch_types=[
        pltpu.VMEM((N_ROWS,), jnp.int32),
        pltpu.VMEM((_PCH,), jnp.int32),
        pltpu.VMEM((N_MEM,), jnp.int32),
    ],
)
def _sc_pos(idx_hbm, pos_hbm, idx_v, pos_b, table_v):
    c = lax.axis_index("c")
    s = lax.axis_index("s")

    @pl.when(jnp.logical_and(c == 0, s == 0))
    def _():
        pltpu.sync_copy(idx_hbm, idx_v)
        lane = lax.iota(jnp.int32, _L)

        def scat_body(g, carry):
            st = pl.multiple_of(g * _L, _L)
            iv = idx_v[pl.ds(st, _L)]
            kvec = lane + g * _L
            for l in range(_L):
                plsc.store_scatter(table_v, [iv], kvec, mask=lane == l)
            return carry

        lax.fori_loop(0, N_ROWS // _L, scat_body, None)

        def chunk_body(ci, carry):
            def lk_body(j, inner):
                st = pl.multiple_of(ci * _PCH + j * _L, _L)
                iv = idx_v[pl.ds(st, _L)]
                dst = pl.multiple_of(j * _L, _L)
                pos_b[pl.ds(dst, _L)] = plsc.load_gather(table_v, [iv])
                return inner

            lax.fori_loop(0, _PCH // _L, lk_body, None)
            pltpu.sync_copy(pos_b, pos_hbm.at[pl.ds(ci * _PCH, _PCH)])
            return carry

        lax.fori_loop(0, N_ROWS // _PCH, chunk_body, None)


@functools.partial(
    pl.kernel,
    mesh=_mesh,
    out_type=jax.ShapeDtypeStruct((N_ROWS, D), jnp.float32),
    scratch_types=[
        pltpu.VMEM((_B_W,), jnp.int32),
        pltpu.VMEM((_CH, D), jnp.float32),
        pltpu.SemaphoreType.DMA,
    ],
)
def _sc_gather(pos_hbm, vn_hbm, out_hbm, pos_v, rows_v, sem):
    c = lax.axis_index("c")
    s = lax.axis_index("s")
    wid = s * 2 + c
    base = wid * _B_W
    pltpu.sync_copy(pos_hbm.at[pl.ds(base, _B_W)], pos_v)

    def body(j, carry):
        off = pl.multiple_of(j * _CH, _CH)
        pltpu.async_copy(
            vn_hbm.at[pos_v.at[pl.ds(off, _CH)]], rows_v, sem
        ).wait()
        pltpu.sync_copy(rows_v, out_hbm.at[pl.ds(base + off, _CH)])
        return carry

    lax.fori_loop(0, _B_W // _CH, body, None)


def kernel(mem, val, idx):
    del mem
    idx32 = idx.astype(jnp.int32)
    vn = _normalize(val)
    pos = _sc_pos(idx32)
    out = _sc_gather(pos, vn)
    return out


# speedup vs baseline: 2.1349x; 2.1349x over previous
import functools

import jax
import jax.numpy as jnp
from jax import lax
from jax.experimental import pallas as pl
from jax.experimental.pallas import tpu as pltpu
from jax.experimental.pallas import tpu_sc as plsc

N_ROWS = 16384
N_MEM = 100000
D = 128
_L = 16
_NW = 32
_B_W = N_ROWS // _NW
_CH = 64
_PCH = 2048


def _norm_body(val_ref, out_ref):
    v = val_ref[...] * (1.0 / 255.0)
    mean = jnp.mean(v, axis=0, keepdims=True)
    mn = jnp.min(v, axis=0, keepdims=True)
    mx = jnp.max(v, axis=0, keepdims=True)
    out_ref[...] = (v - mean) / jnp.abs(mx - mn)


def _normalize(val):
    return pl.pallas_call(
        _norm_body,
        out_shape=jax.ShapeDtypeStruct((N_ROWS, D), jnp.float32),
    )(val)


_mesh = plsc.VectorSubcoreMesh(core_axis_name="c", subcore_axis_name="s")


@functools.partial(
    pl.kernel,
    mesh=_mesh,
    out_type=jax.ShapeDtypeStruct((N_ROWS,), jnp.int32),
    compiler_params=pltpu.CompilerParams(needs_layout_passes=False),
    scratch_types=[
        pltpu.VMEM((N_ROWS,), jnp.int32),
        pltpu.VMEM((_PCH,), jnp.int32),
        pltpu.VMEM((N_MEM,), jnp.int32),
    ],
)
def _sc_pos(idx_hbm, pos_hbm, idx_v, pos_b, table_v):
    c = lax.axis_index("c")
    s = lax.axis_index("s")

    @pl.when(jnp.logical_and(c == 0, s == 0))
    def _():
        pltpu.sync_copy(idx_hbm, idx_v)
        lane = lax.iota(jnp.int32, _L)

        def scat_body(g, carry):
            st = pl.multiple_of(g * _L, _L)
            iv = idx_v[pl.ds(st, _L)]
            kvec = lane + g * _L
            for l in range(_L):
                plsc.store_scatter(table_v, [iv], kvec, mask=lane == l)
            return carry

        lax.fori_loop(0, N_ROWS // _L, scat_body, None)

        def chunk_body(ci, carry):
            def lk_body(j, inner):
                st = pl.multiple_of(ci * _PCH + j * _L, _L)
                iv = idx_v[pl.ds(st, _L)]
                dst = pl.multiple_of(j * _L, _L)
                pos_b[pl.ds(dst, _L)] = plsc.load_gather(table_v, [iv])
                return inner

            lax.fori_loop(0, _PCH // _L, lk_body, None)
            pltpu.sync_copy(pos_b, pos_hbm.at[pl.ds(ci * _PCH, _PCH)])
            return carry

        lax.fori_loop(0, N_ROWS // _PCH, chunk_body, None)


@functools.partial(
    pl.kernel,
    mesh=_mesh,
    out_type=jax.ShapeDtypeStruct((N_ROWS, D), jnp.float32),
    scratch_types=[
        pltpu.VMEM((_B_W,), jnp.int32),
        pltpu.VMEM((_CH, D), jnp.float32),
        pltpu.SemaphoreType.DMA,
    ],
)
def _sc_gather(pos_hbm, vn_hbm, out_hbm, pos_v, rows_v, sem):
    c = lax.axis_index("c")
    s = lax.axis_index("s")
    wid = s * 2 + c
    base = wid * _B_W
    pltpu.sync_copy(pos_hbm.at[pl.ds(base, _B_W)], pos_v)

    def body(j, carry):
        off = pl.multiple_of(j * _CH, _CH)
        pltpu.async_copy(
            vn_hbm.at[pos_v.at[pl.ds(off, _CH)]], rows_v, sem
        ).wait()
        pltpu.sync_copy(rows_v, out_hbm.at[pl.ds(base + off, _CH)])
        return carry

    lax.fori_loop(0, _B_W // _CH, body, None)


def kernel(mem, val, idx):
    del mem
    idx32 = idx.astype(jnp.int32)
    vn = _normalize(val)
    pos = _sc_pos(idx32)
    out = _sc_gather(pos, vn)
    return out


# speedup vs baseline: 2.3554x; 1.1033x over previous
import functools

import jax
import jax.numpy as jnp
from jax import lax
from jax.experimental import pallas as pl
from jax.experimental.pallas import tpu as pltpu
from jax.experimental.pallas import tpu_sc as plsc

N_ROWS = 16384
N_MEM = 100000
D = 128
_L = 16
_NW = 32
_B_W = N_ROWS // _NW
_CH = 64
_NCH = _B_W // _CH
_PCH = 2048
_P = 8
_KS = N_ROWS // _P


def _norm_body(val_ref, out_ref):
    v = val_ref[...] * (1.0 / 255.0)
    mean = jnp.mean(v, axis=0, keepdims=True)
    mn = jnp.min(v, axis=0, keepdims=True)
    mx = jnp.max(v, axis=0, keepdims=True)
    out_ref[...] = (v - mean) / jnp.abs(mx - mn)


def _normalize(val):
    return pl.pallas_call(
        _norm_body,
        out_shape=jax.ShapeDtypeStruct((N_ROWS, D), jnp.float32),
    )(val)


_mesh = plsc.VectorSubcoreMesh(core_axis_name="c", subcore_axis_name="s")


@functools.partial(
    pl.kernel,
    mesh=_mesh,
    out_type=jax.ShapeDtypeStruct((_P, N_ROWS), jnp.int32),
    compiler_params=pltpu.CompilerParams(needs_layout_passes=False),
    scratch_types=[
        pltpu.VMEM((N_ROWS,), jnp.int32),
        pltpu.VMEM((_PCH,), jnp.int32),
        pltpu.VMEM((N_MEM,), jnp.int32),
    ],
)
def _sc_pos(idx_hbm, zeros_hbm, pos_hbm, idx_v, pos_b, table_v):
    c = lax.axis_index("c")
    s = lax.axis_index("s")
    p = c * (_P // 2) + s

    @pl.when(s < _P // 2)
    def _():
        pltpu.sync_copy(zeros_hbm, table_v)
        pltpu.sync_copy(idx_hbm, idx_v)
        lane = lax.iota(jnp.int32, _L)
        kbase = p * _KS

        def scat_body(g, carry):
            st = pl.multiple_of(kbase + g * _L, _L)
            iv = idx_v[pl.ds(st, _L)]
            kvec = lane + st
            for l in range(_L):
                plsc.store_scatter(table_v, [iv], kvec, mask=lane == l)
            return carry

        lax.fori_loop(0, _KS // _L, scat_body, None)

        def chunk_body(ci, carry):
            def lk_body(j, inner):
                st2 = pl.multiple_of(ci * _PCH + j * _L, _L)
                iv = idx_v[pl.ds(st2, _L)]
                dst = pl.multiple_of(j * _L, _L)
                pos_b[pl.ds(dst, _L)] = plsc.load_gather(table_v, [iv])
                return inner

            lax.fori_loop(0, _PCH // _L, lk_body, None)
            pltpu.sync_copy(pos_b, pos_hbm.at[p, pl.ds(ci * _PCH, _PCH)])
            return carry

        lax.fori_loop(0, N_ROWS // _PCH, chunk_body, None)


@functools.partial(
    pl.kernel,
    mesh=_mesh,
    out_type=jax.ShapeDtypeStruct((N_ROWS, D), jnp.float32),
    compiler_params=pltpu.CompilerParams(needs_layout_passes=False),
    scratch_types=[
        pltpu.VMEM((_P, _B_W), jnp.int32),
        pltpu.VMEM((_B_W,), jnp.int32),
        pltpu.VMEM((2, _CH, D), jnp.float32),
        pltpu.SemaphoreType.DMA,
        pltpu.SemaphoreType.DMA,
        pltpu.SemaphoreType.DMA,
        pltpu.SemaphoreType.DMA,
    ],
)
def _sc_gather(pos_hbm, vn_hbm, out_hbm, pos_v, posm_v, rows_v,
               gs0, gs1, ws0, ws1):
    c = lax.axis_index("c")
    s = lax.axis_index("s")
    wid = s * 2 + c
    base = wid * _B_W
    for p in range(_P):
        pltpu.sync_copy(pos_hbm.at[p, pl.ds(base, _B_W)], pos_v.at[p])

    def merge_body(j, carry):
        dst = pl.multiple_of(j * _L, _L)
        m = pos_v[0, pl.ds(dst, _L)]
        for p in range(1, _P):
            m = jnp.maximum(m, pos_v[p, pl.ds(dst, _L)])
        posm_v[pl.ds(dst, _L)] = m
        return carry

    lax.fori_loop(0, _B_W // _L, merge_body, None)

    gsem = (gs0, gs1)
    wsem = (ws0, ws1)

    def g_start(j, b):
        return pltpu.async_copy(
            vn_hbm.at[posm_v.at[pl.ds(j * _CH, _CH)]], rows_v.at[b], gsem[b])

    def w_start(j, b):
        return pltpu.async_copy(
            rows_v.at[b], out_hbm.at[pl.ds(base + j * _CH, _CH)], wsem[b])

    hg = [None] * _NCH
    hw = [None] * _NCH
    hg[0] = g_start(0, 0)
    for j in range(_NCH):
        b = j & 1
        if j + 1 < _NCH:
            if j >= 1:
                hw[j - 1].wait()
            hg[j + 1] = g_start(j + 1, (j + 1) & 1)
        hg[j].wait()
        hw[j] = w_start(j, b)
    hw[_NCH - 2].wait()
    hw[_NCH - 1].wait()


def kernel(mem, val, idx):
    del mem
    idx32 = idx.astype(jnp.int32)
    zeros = jnp.zeros((N_MEM,), jnp.int32)
    vn = _normalize(val)
    pos = _sc_pos(idx32, zeros)
    out = _sc_gather(pos, vn)
    return out


# speedup vs baseline: 2.4857x; 1.0553x over previous
import functools

import jax
import jax.numpy as jnp
from jax import lax
from jax.experimental import pallas as pl
from jax.experimental.pallas import tpu as pltpu
from jax.experimental.pallas import tpu_sc as plsc

N_ROWS = 16384
N_MEM = 100000
D = 128
_L = 16
_NW = 32
_B_W = N_ROWS // _NW
_CH = 128
_NCH = _B_W // _CH
_PCH = 2048
_P = 8
_KS = N_ROWS // _P


def _norm_body(val_ref, out_ref):
    v = val_ref[...] * (1.0 / 255.0)
    mean = jnp.mean(v, axis=0, keepdims=True)
    mn = jnp.min(v, axis=0, keepdims=True)
    mx = jnp.max(v, axis=0, keepdims=True)
    out_ref[...] = (v - mean) / jnp.abs(mx - mn)


def _normalize(val):
    return pl.pallas_call(
        _norm_body,
        out_shape=jax.ShapeDtypeStruct((N_ROWS, D), jnp.float32),
    )(val)


_mesh = plsc.VectorSubcoreMesh(core_axis_name="c", subcore_axis_name="s")


@functools.partial(
    pl.kernel,
    mesh=_mesh,
    out_type=jax.ShapeDtypeStruct((_P, N_ROWS), jnp.int32),
    compiler_params=pltpu.CompilerParams(needs_layout_passes=False),
    scratch_types=[
        pltpu.VMEM((N_ROWS,), jnp.int32),
        pltpu.VMEM((_PCH,), jnp.int32),
        pltpu.VMEM((N_MEM,), jnp.int32),
        pltpu.SemaphoreType.DMA,
        pltpu.SemaphoreType.DMA,
    ],
)
def _sc_pos(idx_hbm, zeros_hbm, pos_hbm, idx_v, pos_b, table_v, sem_a, sem_b):
    c = lax.axis_index("c")
    s = lax.axis_index("s")
    p = c * (_P // 2) + s

    @pl.when(s < _P // 2)
    def _():
        h_t = pltpu.async_copy(zeros_hbm, table_v, sem_a)
        h_i = pltpu.async_copy(idx_hbm, idx_v, sem_b)
        h_i.wait()
        h_t.wait()
        lane = lax.iota(jnp.int32, _L)
        kbase = p * _KS

        def scat_body(g, carry):
            st = pl.multiple_of(kbase + g * _L, _L)
            iv = idx_v[pl.ds(st, _L)]
            kvec = lane + st
            for l in range(_L):
                plsc.store_scatter(table_v, [iv], kvec, mask=lane == l)
            return carry

        lax.fori_loop(0, _KS // _L, scat_body, None)

        def chunk_body(ci, carry):
            def lk_body(j, inner):
                for u in range(8):
                    st2 = pl.multiple_of(
                        ci * _PCH + (j * 8 + u) * _L, _L)
                    iv = idx_v[pl.ds(st2, _L)]
                    dst = pl.multiple_of((j * 8 + u) * _L, _L)
                    pos_b[pl.ds(dst, _L)] = plsc.load_gather(table_v, [iv])
                return inner

            lax.fori_loop(0, _PCH // _L // 8, lk_body, None)
            pltpu.sync_copy(pos_b, pos_hbm.at[p, pl.ds(ci * _PCH, _PCH)])
            return carry

        lax.fori_loop(0, N_ROWS // _PCH, chunk_body, None)


@functools.partial(
    pl.kernel,
    mesh=_mesh,
    out_type=jax.ShapeDtypeStruct((N_ROWS, D), jnp.float32),
    compiler_params=pltpu.CompilerParams(needs_layout_passes=False),
    scratch_types=[
        pltpu.VMEM((_P, _B_W), jnp.int32),
        pltpu.VMEM((_B_W,), jnp.int32),
        pltpu.VMEM((2, _CH, D), jnp.float32),
        pltpu.SemaphoreType.DMA,
        pltpu.SemaphoreType.DMA,
        pltpu.SemaphoreType.DMA,
        pltpu.SemaphoreType.DMA,
    ],
)
def _sc_gather(pos_hbm, vn_hbm, out_hbm, pos_v, posm_v, rows_v,
               gs0, gs1, ws0, ws1):
    c = lax.axis_index("c")
    s = lax.axis_index("s")
    wid = s * 2 + c
    base = wid * _B_W
    for p in range(_P):
        pltpu.sync_copy(pos_hbm.at[p, pl.ds(base, _B_W)], pos_v.at[p])

    def merge_body(j, carry):
        dst = pl.multiple_of(j * _L, _L)
        m = pos_v[0, pl.ds(dst, _L)]
        for p in range(1, _P):
            m = jnp.maximum(m, pos_v[p, pl.ds(dst, _L)])
        posm_v[pl.ds(dst, _L)] = m
        return carry

    lax.fori_loop(0, _B_W // _L, merge_body, None)

    gsem = (gs0, gs1)
    wsem = (ws0, ws1)

    def g_start(j, b):
        return pltpu.async_copy(
            vn_hbm.at[posm_v.at[pl.ds(j * _CH, _CH)]], rows_v.at[b], gsem[b])

    def w_start(j, b):
        return pltpu.async_copy(
            rows_v.at[b], out_hbm.at[pl.ds(base + j * _CH, _CH)], wsem[b])

    hg = [None] * _NCH
    hw = [None] * _NCH
    hg[0] = g_start(0, 0)
    for j in range(_NCH):
        b = j & 1
        if j + 1 < _NCH:
            if j >= 1:
                hw[j - 1].wait()
            hg[j + 1] = g_start(j + 1, (j + 1) & 1)
        hg[j].wait()
        hw[j] = w_start(j, b)
    hw[_NCH - 2].wait()
    hw[_NCH - 1].wait()


def kernel(mem, val, idx):
    del mem
    idx32 = idx.astype(jnp.int32)
    zeros = jnp.zeros((N_MEM,), jnp.int32)
    vn = _normalize(val)
    pos = _sc_pos(idx32, zeros)
    out = _sc_gather(pos, vn)
    return out
